# baseline (speedup 1.0000x reference)
import functools

import jax
import jax.numpy as jnp
from jax.experimental import pallas as pl
from jax.experimental.pallas import tpu as pltpu

NUM_EXPERTS = 8
TOP_K = 2
DIM = 240
BR = 16
BT = BR * 512


def _moe_block(x_ref, gw_ref, gb_ref, ew_ref, eb_ref, o_ref):
    xb = x_ref[...].reshape(BT, DIM)

    logits_t = jax.lax.dot_general(
        gw_ref[...], xb, (((1,), (1,)), ((), ())),
        preferred_element_type=jnp.float32,
    ) + gb_ref[...]

    sub = jax.lax.broadcasted_iota(jnp.int32, (NUM_EXPERTS, BT), 0)
    m1 = jnp.max(logits_t, axis=0, keepdims=True)
    i1 = jnp.min(jnp.where(logits_t == m1, sub, NUM_EXPERTS), axis=0, keepdims=True)
    masked = jnp.where(sub == i1, -jnp.inf, logits_t)
    m2 = jnp.max(masked, axis=0, keepdims=True)
    i2 = jnp.min(jnp.where(masked == m2, sub, NUM_EXPERTS), axis=0, keepdims=True)
    e2 = jnp.exp(m2 - m1)
    w1 = 1.0 / (1.0 + e2)
    w2 = e2 * w1
    wfull_t = jnp.where(sub == i1, w1, 0.0) + jnp.where(sub == i2, w2, 0.0)
    wfull = wfull_t.T

    xb16 = xb.astype(jnp.bfloat16)
    acc = jax.lax.dot_general(
        wfull, eb_ref[...], (((1,), (0,)), ((), ())),
        preferred_element_type=jnp.float32,
    )
    for e in range(NUM_EXPERTS):
        we = wfull[:, e:e + 1]
        ye = jax.lax.dot_general(
            xb16, ew_ref[e], (((1,), (1,)), ((), ())),
            preferred_element_type=jnp.float32,
        )
        acc = acc + we * ye
    o_ref[...] = acc.reshape(BR, 512, DIM)


@jax.jit
def kernel(x, gate_w, gate_b, expert_w, expert_b):
    b, s, d = x.shape
    gb2 = gate_b.reshape(NUM_EXPERTS, 1)
    ew16 = expert_w.astype(jnp.bfloat16)

    return pl.pallas_call(
        _moe_block,
        grid=(b // BR,),
        in_specs=[
            pl.BlockSpec((BR, 512, d), lambda i: (i, 0, 0)),
            pl.BlockSpec((NUM_EXPERTS, d), lambda i: (0, 0)),
            pl.BlockSpec((NUM_EXPERTS, 1), lambda i: (0, 0)),
            pl.BlockSpec((NUM_EXPERTS, d, d), lambda i: (0, 0, 0)),
            pl.BlockSpec((NUM_EXPERTS, d), lambda i: (0, 0)),
        ],
        out_specs=pl.BlockSpec((BR, 512, d), lambda i: (i, 0, 0)),
        out_shape=jax.ShapeDtypeStruct((b, s, d), jnp.float32),
        compiler_params=pltpu.CompilerParams(
            dimension_semantics=("arbitrary",),
        ),
    )(x, gate_w, gb2, ew16, expert_b)


# speedup vs baseline: 1.0118x; 1.0118x over previous
import functools

import jax
import jax.numpy as jnp
from jax.experimental import pallas as pl
from jax.experimental.pallas import tpu as pltpu

NUM_EXPERTS = 8
TOP_K = 2
DIM = 240
BR = 8
BT = BR * 512


def _moe_block(x_ref, gw_ref, gb_ref, ew_ref, eb_ref, o_ref):
    xb = x_ref[...].reshape(BT, DIM)

    logits_t = jax.lax.dot_general(
        gw_ref[...], xb, (((1,), (1,)), ((), ())),
        preferred_element_type=jnp.float32,
    ) + gb_ref[...]

    sub = jax.lax.broadcasted_iota(jnp.int32, (NUM_EXPERTS, BT), 0)
    m1 = jnp.max(logits_t, axis=0, keepdims=True)
    i1 = jnp.min(jnp.where(logits_t == m1, sub, NUM_EXPERTS), axis=0, keepdims=True)
    masked = jnp.where(sub == i1, -jnp.inf, logits_t)
    m2 = jnp.max(masked, axis=0, keepdims=True)
    i2 = jnp.min(jnp.where(masked == m2, sub, NUM_EXPERTS), axis=0, keepdims=True)
    e2 = jnp.exp(m2 - m1)
    w1 = 1.0 / (1.0 + e2)
    w2 = e2 * w1
    wfull_t = jnp.where(sub == i1, w1, 0.0) + jnp.where(sub == i2, w2, 0.0)
    wfull = wfull_t.T

    xb16 = xb.astype(jnp.bfloat16)
    acc = jax.lax.dot_general(
        wfull, eb_ref[...], (((1,), (0,)), ((), ())),
        preferred_element_type=jnp.float32,
    )
    for e in range(NUM_EXPERTS):
        we = wfull[:, e:e + 1]
        ye = jax.lax.dot_general(
            xb16, ew_ref[e], (((1,), (1,)), ((), ())),
            preferred_element_type=jnp.float32,
        )
        acc = acc + we * ye
    o_ref[...] = acc.reshape(BR, 512, DIM)


@jax.jit
def kernel(x, gate_w, gate_b, expert_w, expert_b):
    b, s, d = x.shape
    gb2 = gate_b.reshape(NUM_EXPERTS, 1)
    ew16 = expert_w.astype(jnp.bfloat16)

    return pl.pallas_call(
        _moe_block,
        grid=(b // BR,),
        in_specs=[
            pl.BlockSpec((BR, 512, d), lambda i: (i, 0, 0)),
            pl.BlockSpec((NUM_EXPERTS, d), lambda i: (0, 0)),
            pl.BlockSpec((NUM_EXPERTS, 1), lambda i: (0, 0)),
            pl.BlockSpec((NUM_EXPERTS, d, d), lambda i: (0, 0, 0)),
            pl.BlockSpec((NUM_EXPERTS, d), lambda i: (0, 0)),
        ],
        out_specs=pl.BlockSpec((BR, 512, d), lambda i: (i, 0, 0)),
        out_shape=jax.ShapeDtypeStruct((b, s, d), jnp.float32),
        compiler_params=pltpu.CompilerParams(
            dimension_semantics=("arbitrary",),
        ),
    )(x, gate_w, gb2, ew16, expert_b)


# speedup vs baseline: 1.8594x; 1.8378x over previous
import functools

import jax
import jax.numpy as jnp
from jax.experimental import pallas as pl
from jax.experimental.pallas import tpu as pltpu

NUM_EXPERTS = 8
TOP_K = 2
DIM = 240
SEQ = 512
BR = 8


def _moe_block(x_ref, gw_ref, gb_ref, ew_ref, eb_ref, o_ref):
    for r in range(BR):
        xbt = x_ref[r]

        logits_t = jax.lax.dot_general(
            gw_ref[...], xbt, (((1,), (0,)), ((), ())),
            preferred_element_type=jnp.float32,
        ) + gb_ref[...]

        sub = jax.lax.broadcasted_iota(jnp.int32, (NUM_EXPERTS, SEQ), 0)
        m1 = jnp.max(logits_t, axis=0, keepdims=True)
        i1 = jnp.min(jnp.where(logits_t == m1, sub, NUM_EXPERTS), axis=0,
                     keepdims=True)
        masked = jnp.where(sub == i1, -jnp.inf, logits_t)
        m2 = jnp.max(masked, axis=0, keepdims=True)
        i2 = jnp.min(jnp.where(masked == m2, sub, NUM_EXPERTS), axis=0,
                     keepdims=True)
        e2 = jnp.exp(m2 - m1)
        w1 = 1.0 / (1.0 + e2)
        w2 = e2 * w1
        wt = jnp.where(sub == i1, w1, 0.0) + jnp.where(sub == i2, w2, 0.0)

        acc = jax.lax.dot_general(
            eb_ref[...], wt, (((0,), (0,)), ((), ())),
            preferred_element_type=jnp.float32,
        )
        for e in range(NUM_EXPERTS):
            yet = jax.lax.dot_general(
                ew_ref[e], xbt, (((1,), (0,)), ((), ())),
                preferred_element_type=jnp.float32,
            )
            acc = acc + wt[e:e + 1, :] * yet
        o_ref[r] = acc


@jax.jit
def kernel(x, gate_w, gate_b, expert_w, expert_b):
    b, s, d = x.shape
    xt = jnp.transpose(x, (0, 2, 1))
    gb2 = gate_b.reshape(NUM_EXPERTS, 1)

    out_t = pl.pallas_call(
        _moe_block,
        grid=(b // BR,),
        in_specs=[
            pl.BlockSpec((BR, d, s), lambda i: (i, 0, 0)),
            pl.BlockSpec((NUM_EXPERTS, d), lambda i: (0, 0)),
            pl.BlockSpec((NUM_EXPERTS, 1), lambda i: (0, 0)),
            pl.BlockSpec((NUM_EXPERTS, d, d), lambda i: (0, 0, 0)),
            pl.BlockSpec((NUM_EXPERTS, d), lambda i: (0, 0)),
        ],
        out_specs=pl.BlockSpec((BR, d, s), lambda i: (i, 0, 0)),
        out_shape=jax.ShapeDtypeStruct((b, d, s), jnp.float32),
        compiler_params=pltpu.CompilerParams(
            dimension_semantics=("arbitrary",),
        ),
    )(xt, gate_w, gb2, expert_w, expert_b)
    return jnp.transpose(out_t, (0, 2, 1))
